# baseline (speedup 1.0000x reference)
import jax
import jax.numpy as jnp
from jax.experimental import pallas as pl

_B, _N, _D = 8, 4096, 768
_C = 16
_NUM_CLASSES = 2
_THR = 0.8


def _rdd_body(c_ref, x_ref, w_ref, b_ref, o_ref):
    bidx = pl.program_id(0)
    x = x_ref[0]
    cid = jax.lax.broadcasted_iota(jnp.int32, (_C, _N), 0)
    maskb = (c_ref[0] == cid).astype(jnp.bfloat16)
    parts = []
    for h in range(4):
        xh = x[:, h * (_D // 4):(h + 1) * (_D // 4)]
        x_hi = xh.astype(jnp.bfloat16)
        x_lo = (xh - x_hi.astype(jnp.float32)).astype(jnp.bfloat16)
        parts.append(
            jnp.dot(maskb, x_hi, preferred_element_type=jnp.float32)
            + jnp.dot(maskb, x_lo, preferred_element_type=jnp.float32)
        )
    sums = jnp.concatenate(parts, axis=1)
    counts = jnp.sum(
        maskb.astype(jnp.float32), axis=1, keepdims=True
    )
    feats = sums / jnp.maximum(counts, 1.0)
    logits = (
        jnp.dot(feats, w_ref[...], preferred_element_type=jnp.float32)
        + b_ref[0]
    )
    d = logits[:, 1:2] - logits[:, 0:1]
    dmax = jnp.max(d)
    dmin = jnp.min(d)
    use_min = jax.nn.sigmoid(dmax) < _THR
    target = jnp.where(use_min, dmin, dmax)
    idxs = jax.lax.broadcasted_iota(jnp.int32, (_C, 1), 0)
    sel = jnp.min(jnp.where(d == target, idxs, _C))
    selmask = (idxs == sel).astype(jnp.float32)
    out = jnp.sum(selmask * logits, axis=0, keepdims=True)
    o_ref[pl.ds(bidx, 1), :] = out


@jax.jit
def _run(inst_feat, clusters, W, b2):
    return pl.pallas_call(
        _rdd_body,
        grid=(_B,),
        in_specs=[
            pl.BlockSpec((1, 1, _N), lambda i: (i, 0, 0)),
            pl.BlockSpec((1, _N, _D), lambda i: (i, 0, 0)),
            pl.BlockSpec((_D, _NUM_CLASSES), lambda i: (0, 0)),
            pl.BlockSpec((1, _NUM_CLASSES), lambda i: (0, 0)),
        ],
        out_specs=pl.BlockSpec((_B, _NUM_CLASSES), lambda i: (0, 0)),
        out_shape=jax.ShapeDtypeStruct((_B, _NUM_CLASSES), jnp.float32),
    )(clusters, inst_feat, W, b2)


def kernel(inst_feat, clusters_idcs, W, b):
    clusters = clusters_idcs.astype(jnp.int32).reshape(_B, 1, _N)
    b2 = b.reshape(1, _NUM_CLASSES).astype(jnp.float32)
    return _run(inst_feat, clusters, W, b2)


# speedup vs baseline: 1.0685x; 1.0685x over previous
import jax
import jax.numpy as jnp
from jax.experimental import pallas as pl

_B, _N, _D = 8, 4096, 768
_C = 16
_NUM_CLASSES = 2
_THR = 0.8


def _rdd_body(c_ref, x_ref, w_ref, b_ref, o_ref):
    bidx = pl.program_id(0)
    x = x_ref[0]
    cid = jax.lax.broadcasted_iota(jnp.int32, (_C, _N), 0)
    maskb = (c_ref[0] == cid).astype(jnp.bfloat16)
    parts = []
    for h in range(2):
        xh = x[:, h * (_D // 2):(h + 1) * (_D // 2)]
        x_hi = xh.astype(jnp.bfloat16)
        x_lo = (xh - x_hi.astype(jnp.float32)).astype(jnp.bfloat16)
        parts.append(
            jnp.dot(maskb, x_hi, preferred_element_type=jnp.float32)
            + jnp.dot(maskb, x_lo, preferred_element_type=jnp.float32)
        )
    sums = jnp.concatenate(parts, axis=1)
    counts = jnp.sum(
        maskb.astype(jnp.float32), axis=1, keepdims=True
    )
    feats = sums / jnp.maximum(counts, 1.0)
    logits = (
        jnp.dot(feats, w_ref[...], preferred_element_type=jnp.float32)
        + b_ref[0]
    )
    d = logits[:, 1:2] - logits[:, 0:1]
    dmax = jnp.max(d)
    dmin = jnp.min(d)
    use_min = jax.nn.sigmoid(dmax) < _THR
    target = jnp.where(use_min, dmin, dmax)
    idxs = jax.lax.broadcasted_iota(jnp.int32, (_C, 1), 0)
    sel = jnp.min(jnp.where(d == target, idxs, _C))
    selmask = (idxs == sel).astype(jnp.float32)
    out = jnp.sum(selmask * logits, axis=0, keepdims=True)
    o_ref[pl.ds(bidx, 1), :] = out


@jax.jit
def _run(inst_feat, clusters, W, b2):
    return pl.pallas_call(
        _rdd_body,
        grid=(_B,),
        in_specs=[
            pl.BlockSpec((1, 1, _N), lambda i: (i, 0, 0)),
            pl.BlockSpec((1, _N, _D), lambda i: (i, 0, 0)),
            pl.BlockSpec((_D, _NUM_CLASSES), lambda i: (0, 0)),
            pl.BlockSpec((1, _NUM_CLASSES), lambda i: (0, 0)),
        ],
        out_specs=pl.BlockSpec((_B, _NUM_CLASSES), lambda i: (0, 0)),
        out_shape=jax.ShapeDtypeStruct((_B, _NUM_CLASSES), jnp.float32),
    )(clusters, inst_feat, W, b2)


def kernel(inst_feat, clusters_idcs, W, b):
    clusters = clusters_idcs.astype(jnp.int32).reshape(_B, 1, _N)
    b2 = b.reshape(1, _NUM_CLASSES).astype(jnp.float32)
    return _run(inst_feat, clusters, W, b2)
